# Initial kernel scaffold; baseline (speedup 1.0000x reference)
import jax
import jax.numpy as jnp
from jax.experimental import pallas as pl


def kernel(t, beta, alpha):
    raise NotImplementedError("write your pallas kernel here")


# speedup vs baseline: 8.8265x; 8.8265x over previous
import functools

import jax
import jax.numpy as jnp
from jax import lax
from jax.experimental import pallas as pl
from jax.experimental.pallas import tpu as pltpu
from jax.experimental.pallas import tpu_sc as plsc

NUM_T = 1000
BATCH = 16384
NC = 2
NS = 16
NW = NC * NS
LANES = 16
B_PER_W = BATCH // NW


@functools.partial(
    pl.kernel,
    out_type=(
        jax.ShapeDtypeStruct((BATCH,), jnp.float32),
        jax.ShapeDtypeStruct((BATCH,), jnp.float32),
    ),
    mesh=plsc.VectorSubcoreMesh(core_axis_name="c", subcore_axis_name="s"),
    scratch_types=[
        pltpu.VMEM((B_PER_W,), jnp.int32),
        pltpu.VMEM((NUM_T,), jnp.float32),
        pltpu.VMEM((NUM_T,), jnp.float32),
        pltpu.VMEM((B_PER_W,), jnp.float32),
        pltpu.VMEM((B_PER_W,), jnp.float32),
        pltpu.SemaphoreType.DMA,
    ],
    compiler_params=pltpu.CompilerParams(needs_layout_passes=False),
)
def _ddpm_lookup(t_hbm, beta_hbm, alpha_hbm, beta_out, alpha_out,
                 idx_v, tbl_b, tbl_a, res_b, res_a, sem):
    wid = lax.axis_index("s") * NC + lax.axis_index("c")
    base = wid * B_PER_W

    cp_idx = pltpu.async_copy(t_hbm.at[pl.ds(base, B_PER_W)], idx_v, sem)
    cp_b = pltpu.async_copy(beta_hbm, tbl_b, sem)
    cp_a = pltpu.async_copy(alpha_hbm, tbl_a, sem)
    cp_idx.wait()
    cp_b.wait()
    cp_a.wait()

    for j in range(B_PER_W // LANES):
        sl = pl.ds(j * LANES, LANES)
        idx = idx_v[sl]
        res_b[sl] = plsc.load_gather(tbl_b, [idx])
        res_a[sl] = plsc.load_gather(tbl_a, [idx])

    out_sl = pl.ds(base, B_PER_W)
    cp_ob = pltpu.async_copy(res_b, beta_out.at[out_sl], sem)
    cp_oa = pltpu.async_copy(res_a, alpha_out.at[out_sl], sem)
    cp_ob.wait()
    cp_oa.wait()


def kernel(t, beta, alpha):
    return _ddpm_lookup(t, beta, alpha)
